# baseline (speedup 1.0000x reference)
import functools

import jax
import jax.numpy as jnp
import numpy as np
from jax import lax
from jax.experimental import pallas as pl
from jax.experimental.pallas import tpu as pltpu
from jax.experimental.pallas import tpu_sc as plsc

N = 10000
D = 128
H = 64
E = 320000

NC = 2
NS = 16
NW = NC * NS

C = 128
NPAD = 10240
EPW = 10240
E_PAD = NW * EPW
NCHUNK = EPW // C
ROWS_PT = NPAD // NS

def _deg_body(dst_hbm, zeros1_hbm, out_hbm, idx_dst, ones_v, deg_sh):
    c = lax.axis_index("c")
    s = lax.axis_index("s")
    w = c * NS + s
    for i in range(C // 16):
        ones_v[pl.ds(i * 16, 16)] = jnp.ones((16,), jnp.float32)
    pltpu.sync_copy(dst_hbm.at[w], idx_dst)
    pltpu.sync_copy(
        zeros1_hbm.at[pl.ds(s * ROWS_PT, ROWS_PT)],
        deg_sh.at[pl.ds(s * ROWS_PT, ROWS_PT)],
    )
    plsc.subcore_barrier()

    def body(i, carry):
        pltpu.sync_copy(ones_v, deg_sh.at[idx_dst.at[i]], add=True)
        return carry

    lax.fori_loop(0, NCHUNK, body, 0)
    plsc.subcore_barrier()
    pltpu.sync_copy(
        deg_sh.at[pl.ds(s * ROWS_PT, ROWS_PT)],
        out_hbm.at[pl.ds(c * NPAD + s * ROWS_PT, ROWS_PT)],
    )


NB = 2


def _agg_body(src_hbm, dst_hbm, g_hbm, zerosh_hbm, out_hbm,
              idx_src, idx_dst, rows, acc_sh, g_sh, gsems, ssems):
    c = lax.axis_index("c")
    s = lax.axis_index("s")
    w = c * NS + s
    pltpu.sync_copy(src_hbm.at[w], idx_src)
    pltpu.sync_copy(dst_hbm.at[w], idx_dst)

    @pl.when(s < NS - 1)
    def _():
        pltpu.sync_copy(
            g_hbm.at[pl.ds(s * ROWS_PT, ROWS_PT)],
            g_sh.at[pl.ds(s * ROWS_PT, ROWS_PT)],
        )

    @pl.when(s == NS - 1)
    def _():
        pltpu.sync_copy(
            g_hbm.at[pl.ds((NS - 1) * ROWS_PT, N - (NS - 1) * ROWS_PT)],
            g_sh.at[pl.ds((NS - 1) * ROWS_PT, N - (NS - 1) * ROWS_PT)],
        )

    @pl.when(c == 0)
    def _():
        @pl.when(s < NS - 1)
        def _():
            pltpu.sync_copy(
                g_hbm.at[pl.ds(s * ROWS_PT, ROWS_PT)],
                acc_sh.at[pl.ds(s * ROWS_PT, ROWS_PT)],
            )

        @pl.when(s == NS - 1)
        def _():
            pltpu.sync_copy(
                g_hbm.at[pl.ds((NS - 1) * ROWS_PT, N - (NS - 1) * ROWS_PT)],
                acc_sh.at[pl.ds((NS - 1) * ROWS_PT, N - (NS - 1) * ROWS_PT)],
            )
            pltpu.sync_copy(
                zerosh_hbm.at[pl.ds(0, NPAD - N)],
                acc_sh.at[pl.ds(N, NPAD - N)],
            )

    @pl.when(c != 0)
    def _():
        pltpu.sync_copy(
            zerosh_hbm.at[pl.ds(s * ROWS_PT, ROWS_PT)],
            acc_sh.at[pl.ds(s * ROWS_PT, ROWS_PT)],
        )

    plsc.subcore_barrier()

    for b in range(NB):
        pltpu.async_copy(g_sh.at[idx_src.at[b]], rows.at[b], gsems.at[b])

    def body(j, carry):
        base = j * NB
        for b in range(NB):
            pltpu.make_async_copy(
                g_sh.at[idx_src.at[base + b]], rows.at[b], gsems.at[b]
            ).wait()
            pltpu.async_copy(
                rows.at[b], acc_sh.at[idx_dst.at[base + b]], ssems.at[b], add=True
            )
        for b in range(NB):
            pltpu.make_async_copy(
                rows.at[b], acc_sh.at[idx_dst.at[base + b]], ssems.at[b]
            ).wait()
            nxt = base + NB + b

            @pl.when(nxt < NCHUNK)
            def _():
                pltpu.async_copy(
                    g_sh.at[idx_src.at[nxt]], rows.at[b], gsems.at[b]
                )

        return carry

    lax.fori_loop(0, NCHUNK // NB, body, 0)
    plsc.subcore_barrier()
    pltpu.sync_copy(
        acc_sh.at[pl.ds(s * ROWS_PT, ROWS_PT)],
        out_hbm.at[pl.ds(c * NPAD + s * ROWS_PT, ROWS_PT)],
    )


def _matmul_body(x_ref, w_ref, h_ref):
    h_ref[...] = jnp.dot(
        x_ref[...], w_ref[...], preferred_element_type=jnp.float32
    )


def _scale_body(degp_ref, h_ref, g_ref, dis_ref):
    deg = degp_ref[0:1, :] + degp_ref[1:2, :] + 1.0
    dis = lax.rsqrt(deg)
    dis_ref[...] = dis
    dis_col = jnp.transpose(dis[:, :N])
    g_ref[...] = h_ref[...] * dis_col


def _final_body(accp_ref, dis_ref, b_ref, w2_ref, b2_ref, out_ref):
    acc = accp_ref[0, :N, :] + accp_ref[1, :N, :]
    dis_col = jnp.transpose(dis_ref[:, :N])
    t = jnp.maximum(acc * dis_col + b_ref[...], 0.0)
    val = jnp.sum(t * w2_ref[...], keepdims=True) * (1.0 / N) + b2_ref[...]
    out_ref[...] = val


@functools.cache
def _sc_kernels():
    mesh = plsc.VectorSubcoreMesh(
        core_axis_name="c", subcore_axis_name="s", num_cores=NC, num_subcores=NS
    )
    deg_kernel = pl.kernel(
        _deg_body,
        out_type=jax.ShapeDtypeStruct((NC * NPAD,), jnp.float32),
        mesh=mesh,
        scratch_types=[
            pltpu.VMEM((NCHUNK, C), jnp.int32),
            pltpu.VMEM((C,), jnp.float32),
            pltpu.VMEM_SHARED((NPAD,), jnp.float32),
        ],
    )
    agg_kernel = pl.kernel(
        _agg_body,
        out_type=jax.ShapeDtypeStruct((NC * NPAD, H), jnp.float32),
        mesh=mesh,
        compiler_params=pltpu.CompilerParams(use_tc_tiling_on_sc=False),
        scratch_types=[
            pltpu.VMEM((NCHUNK, C), jnp.int32),
            pltpu.VMEM((NCHUNK, C), jnp.int32),
            pltpu.VMEM((NB, C, H), jnp.float32),
            pltpu.VMEM_SHARED((NPAD, H), jnp.float32),
            pltpu.VMEM_SHARED((N, H), jnp.float32),
            pltpu.SemaphoreType.DMA((NB,)),
            pltpu.SemaphoreType.DMA((NB,)),
        ],
    )
    return deg_kernel, agg_kernel


_AR = np.arange(E_PAD - E, dtype=np.int32)
_PAD_SRC = jnp.asarray(_AR % N)
_PAD_DST = jnp.asarray(N + (_AR % (NPAD - N)))


def kernel(x, edge_index, W, b, W2, b2):
    src = edge_index[0].astype(jnp.int32)
    dst = edge_index[1].astype(jnp.int32)
    src_p = jnp.concatenate([src, _PAD_SRC]).reshape(NW, NCHUNK, C)
    dst_p = jnp.concatenate([dst, _PAD_DST]).reshape(NW, NCHUNK, C)

    zeros1 = jnp.zeros((NPAD,), jnp.float32)
    zerosh = jnp.zeros((NPAD, H), jnp.float32)

    deg_kernel, agg_kernel = _sc_kernels()
    degp = deg_kernel(dst_p, zeros1)
    degp2 = degp.reshape(NC, NPAD)

    h = pl.pallas_call(
        _matmul_body,
        out_shape=jax.ShapeDtypeStruct((N, H), jnp.float32),
    )(x, W)

    g, dis = pl.pallas_call(
        _scale_body,
        out_shape=[
            jax.ShapeDtypeStruct((N, H), jnp.float32),
            jax.ShapeDtypeStruct((1, NPAD), jnp.float32),
        ],
    )(degp2, h)

    accp = agg_kernel(src_p, dst_p, g, zerosh)
    accp3 = accp.reshape(NC, NPAD, H)

    pooled = pl.pallas_call(
        _final_body,
        out_shape=jax.ShapeDtypeStruct((1, 1), jnp.float32),
    )(accp3, dis, b.reshape(1, H), W2.reshape(1, H), b2.reshape(1, 1))
    return pooled


# speedup vs baseline: 1.1867x; 1.1867x over previous
import functools

import jax
import jax.numpy as jnp
import numpy as np
from jax import lax
from jax.experimental import pallas as pl
from jax.experimental.pallas import tpu as pltpu
from jax.experimental.pallas import tpu_sc as plsc

N = 10000
D = 128
H = 64
E = 320000

NC = 2
NS = 16
NW = NC * NS

C = 128
NPAD = 10240
EPW = 10240
E_PAD = NW * EPW
NCHUNK = EPW // C
ROWS_PT = NPAD // NS

def _deg_body(dst_hbm, zeros1_hbm, out_hbm, idx_dst, ones_v, deg_sh):
    c = lax.axis_index("c")
    s = lax.axis_index("s")
    w = c * NS + s
    for i in range(C // 16):
        ones_v[pl.ds(i * 16, 16)] = jnp.ones((16,), jnp.float32)
    pltpu.sync_copy(dst_hbm.at[w], idx_dst)
    pltpu.sync_copy(
        zeros1_hbm.at[pl.ds(s * ROWS_PT, ROWS_PT)],
        deg_sh.at[pl.ds(s * ROWS_PT, ROWS_PT)],
    )
    plsc.subcore_barrier()

    def body(i, carry):
        pltpu.sync_copy(ones_v, deg_sh.at[idx_dst.at[i]], add=True)
        return carry

    lax.fori_loop(0, NCHUNK, body, 0)
    plsc.subcore_barrier()
    pltpu.sync_copy(
        deg_sh.at[pl.ds(s * ROWS_PT, ROWS_PT)],
        out_hbm.at[pl.ds(c * NPAD + s * ROWS_PT, ROWS_PT)],
    )


NB = 8


def _agg_body(src_hbm, dst_hbm, g_hbm, zerosh_hbm, out_hbm,
              idx_src, idx_dst, rows, acc_sh, gsems, ssems):
    c = lax.axis_index("c")
    s = lax.axis_index("s")
    w = c * NS + s
    pltpu.sync_copy(src_hbm.at[w], idx_src)
    pltpu.sync_copy(dst_hbm.at[w], idx_dst)

    @pl.when(c == 0)
    def _():
        @pl.when(s < NS - 1)
        def _():
            pltpu.sync_copy(
                g_hbm.at[pl.ds(s * ROWS_PT, ROWS_PT)],
                acc_sh.at[pl.ds(s * ROWS_PT, ROWS_PT)],
            )

        @pl.when(s == NS - 1)
        def _():
            pltpu.sync_copy(
                g_hbm.at[pl.ds((NS - 1) * ROWS_PT, N - (NS - 1) * ROWS_PT)],
                acc_sh.at[pl.ds((NS - 1) * ROWS_PT, N - (NS - 1) * ROWS_PT)],
            )
            pltpu.sync_copy(
                zerosh_hbm.at[pl.ds(0, NPAD - N)],
                acc_sh.at[pl.ds(N, NPAD - N)],
            )

    @pl.when(c != 0)
    def _():
        pltpu.sync_copy(
            zerosh_hbm.at[pl.ds(s * ROWS_PT, ROWS_PT)],
            acc_sh.at[pl.ds(s * ROWS_PT, ROWS_PT)],
        )

    plsc.subcore_barrier()

    for b in range(NB):
        pltpu.async_copy(g_hbm.at[idx_src.at[b]], rows.at[b], gsems.at[b])

    def body(j, carry):
        base = j * NB
        for b in range(NB):
            pltpu.make_async_copy(
                g_hbm.at[idx_src.at[base + b]], rows.at[b], gsems.at[b]
            ).wait()
            pltpu.async_copy(
                rows.at[b], acc_sh.at[idx_dst.at[base + b]], ssems.at[b], add=True
            )
        for b in range(NB):
            pltpu.make_async_copy(
                rows.at[b], acc_sh.at[idx_dst.at[base + b]], ssems.at[b]
            ).wait()
            nxt = base + NB + b

            @pl.when(nxt < NCHUNK)
            def _():
                pltpu.async_copy(
                    g_hbm.at[idx_src.at[nxt]], rows.at[b], gsems.at[b]
                )

        return carry

    lax.fori_loop(0, NCHUNK // NB, body, 0)
    plsc.subcore_barrier()
    pltpu.sync_copy(
        acc_sh.at[pl.ds(s * ROWS_PT, ROWS_PT)],
        out_hbm.at[pl.ds(c * NPAD + s * ROWS_PT, ROWS_PT)],
    )


def _matmul_body(x_ref, w_ref, h_ref):
    h_ref[...] = jnp.dot(
        x_ref[...], w_ref[...], preferred_element_type=jnp.float32
    )


def _scale_body(degp_ref, h_ref, g_ref, dis_ref):
    deg = degp_ref[0:1, :] + degp_ref[1:2, :] + 1.0
    dis = lax.rsqrt(deg)
    dis_ref[...] = dis
    dis_col = jnp.transpose(dis[:, :N])
    g_ref[...] = h_ref[...] * dis_col


def _final_body(accp_ref, dis_ref, b_ref, w2_ref, b2_ref, out_ref):
    acc = accp_ref[0, :N, :] + accp_ref[1, :N, :]
    dis_col = jnp.transpose(dis_ref[:, :N])
    t = jnp.maximum(acc * dis_col + b_ref[...], 0.0)
    val = jnp.sum(t * w2_ref[...], keepdims=True) * (1.0 / N) + b2_ref[...]
    out_ref[...] = val


@functools.cache
def _sc_kernels():
    mesh = plsc.VectorSubcoreMesh(
        core_axis_name="c", subcore_axis_name="s", num_cores=NC, num_subcores=NS
    )
    deg_kernel = pl.kernel(
        _deg_body,
        out_type=jax.ShapeDtypeStruct((NC * NPAD,), jnp.float32),
        mesh=mesh,
        scratch_types=[
            pltpu.VMEM((NCHUNK, C), jnp.int32),
            pltpu.VMEM((C,), jnp.float32),
            pltpu.VMEM_SHARED((NPAD,), jnp.float32),
        ],
    )
    agg_kernel = pl.kernel(
        _agg_body,
        out_type=jax.ShapeDtypeStruct((NC * NPAD, H), jnp.float32),
        mesh=mesh,
        compiler_params=pltpu.CompilerParams(use_tc_tiling_on_sc=False),
        scratch_types=[
            pltpu.VMEM((NCHUNK, C), jnp.int32),
            pltpu.VMEM((NCHUNK, C), jnp.int32),
            pltpu.VMEM((NB, C, H), jnp.float32),
            pltpu.VMEM_SHARED((NPAD, H), jnp.float32),
            pltpu.SemaphoreType.DMA((NB,)),
            pltpu.SemaphoreType.DMA((NB,)),
        ],
    )
    return deg_kernel, agg_kernel


_AR = np.arange(E_PAD - E, dtype=np.int32)
_PAD_SRC = jnp.asarray(_AR % N)
_PAD_DST = jnp.asarray(N + (_AR % (NPAD - N)))


def kernel(x, edge_index, W, b, W2, b2):
    src = edge_index[0].astype(jnp.int32)
    dst = edge_index[1].astype(jnp.int32)
    src_p = jnp.concatenate([src, _PAD_SRC]).reshape(NW, NCHUNK, C)
    dst_p = jnp.concatenate([dst, _PAD_DST]).reshape(NW, NCHUNK, C)

    zeros1 = jnp.zeros((NPAD,), jnp.float32)
    zerosh = jnp.zeros((NPAD, H), jnp.float32)

    deg_kernel, agg_kernel = _sc_kernels()
    degp = deg_kernel(dst_p, zeros1)
    degp2 = degp.reshape(NC, NPAD)

    h = pl.pallas_call(
        _matmul_body,
        out_shape=jax.ShapeDtypeStruct((N, H), jnp.float32),
    )(x, W)

    g, dis = pl.pallas_call(
        _scale_body,
        out_shape=[
            jax.ShapeDtypeStruct((N, H), jnp.float32),
            jax.ShapeDtypeStruct((1, NPAD), jnp.float32),
        ],
    )(degp2, h)

    accp = agg_kernel(src_p, dst_p, g, zerosh)
    accp3 = accp.reshape(NC, NPAD, H)

    pooled = pl.pallas_call(
        _final_body,
        out_shape=jax.ShapeDtypeStruct((1, 1), jnp.float32),
    )(accp3, dis, b.reshape(1, H), W2.reshape(1, H), b2.reshape(1, 1))
    return pooled


# speedup vs baseline: 1.1898x; 1.0026x over previous
import functools

import jax
import jax.numpy as jnp
import numpy as np
from jax import lax
from jax.experimental import pallas as pl
from jax.experimental.pallas import tpu as pltpu
from jax.experimental.pallas import tpu_sc as plsc

N = 10000
D = 128
H = 64
E = 320000

NC = 2
NS = 16
NW = NC * NS

C = 128
NPAD = 10240
EPW = 10240
E_PAD = NW * EPW
NCHUNK = EPW // C
ROWS_PT = NPAD // NS

def _deg_body(dst_hbm, zeros1_hbm, out_hbm, idx_dst, ones_v, deg_sh):
    c = lax.axis_index("c")
    s = lax.axis_index("s")
    w = c * NS + s
    for i in range(C // 16):
        ones_v[pl.ds(i * 16, 16)] = jnp.ones((16,), jnp.float32)
    pltpu.sync_copy(dst_hbm.at[w], idx_dst)
    pltpu.sync_copy(
        zeros1_hbm.at[pl.ds(s * ROWS_PT, ROWS_PT)],
        deg_sh.at[pl.ds(s * ROWS_PT, ROWS_PT)],
    )
    plsc.subcore_barrier()

    def body(i, carry):
        pltpu.sync_copy(ones_v, deg_sh.at[idx_dst.at[i]], add=True)
        return carry

    lax.fori_loop(0, NCHUNK, body, 0)
    plsc.subcore_barrier()
    pltpu.sync_copy(
        deg_sh.at[pl.ds(s * ROWS_PT, ROWS_PT)],
        out_hbm.at[pl.ds(c * NPAD + s * ROWS_PT, ROWS_PT)],
    )


NB = 8


def _agg_body(src_hbm, dst_hbm, g_hbm, zerosh_hbm, out_hbm,
              idx_src, idx_dst, rows, acc_sh, gsems, ssems):
    c = lax.axis_index("c")
    s = lax.axis_index("s")
    w = c * NS + s
    pltpu.sync_copy(src_hbm.at[w], idx_src)
    pltpu.sync_copy(dst_hbm.at[w], idx_dst)

    @pl.when(c == 0)
    def _():
        @pl.when(s < NS - 1)
        def _():
            pltpu.sync_copy(
                g_hbm.at[pl.ds(s * ROWS_PT, ROWS_PT)],
                acc_sh.at[pl.ds(s * ROWS_PT, ROWS_PT)],
            )

        @pl.when(s == NS - 1)
        def _():
            pltpu.sync_copy(
                g_hbm.at[pl.ds((NS - 1) * ROWS_PT, N - (NS - 1) * ROWS_PT)],
                acc_sh.at[pl.ds((NS - 1) * ROWS_PT, N - (NS - 1) * ROWS_PT)],
            )
            pltpu.sync_copy(
                zerosh_hbm.at[pl.ds(0, NPAD - N)],
                acc_sh.at[pl.ds(N, NPAD - N)],
            )

    @pl.when(c != 0)
    def _():
        pltpu.sync_copy(
            zerosh_hbm.at[pl.ds(s * ROWS_PT, ROWS_PT)],
            acc_sh.at[pl.ds(s * ROWS_PT, ROWS_PT)],
        )

    plsc.subcore_barrier()

    for b in range(NB):
        pltpu.async_copy(g_hbm.at[idx_src.at[b]], rows.at[b], gsems.at[b])

    def body(j, carry):
        base = j * NB
        for b in range(NB):
            pltpu.make_async_copy(
                g_hbm.at[idx_src.at[base + b]], rows.at[b], gsems.at[b]
            ).wait()
            pltpu.async_copy(
                rows.at[b], acc_sh.at[idx_dst.at[base + b]], ssems.at[b], add=True
            )
        for b in range(NB):
            pltpu.make_async_copy(
                rows.at[b], acc_sh.at[idx_dst.at[base + b]], ssems.at[b]
            ).wait()
            nxt = base + NB + b

            @pl.when(nxt < NCHUNK)
            def _():
                pltpu.async_copy(
                    g_hbm.at[idx_src.at[nxt]], rows.at[b], gsems.at[b]
                )

        return carry

    lax.fori_loop(0, NCHUNK // NB, body, 0)
    plsc.subcore_barrier()
    pltpu.sync_copy(
        acc_sh.at[pl.ds(s * ROWS_PT, ROWS_PT)],
        out_hbm.at[pl.ds(c * NPAD + s * ROWS_PT, ROWS_PT)],
    )


def _mmscale_body(x_ref, w_ref, degp_ref, g_ref, dis_ref):
    h = jnp.dot(x_ref[...], w_ref[...], preferred_element_type=jnp.float32)
    deg = degp_ref[0:1, :] + degp_ref[1:2, :] + 1.0
    dis = lax.rsqrt(deg)
    dis_ref[...] = dis
    dis_col = jnp.transpose(dis[:, :N])
    g_ref[...] = h * dis_col


def _final_body(accp_ref, dis_ref, b_ref, w2_ref, b2_ref, out_ref):
    acc = accp_ref[0, :N, :] + accp_ref[1, :N, :]
    dis_col = jnp.transpose(dis_ref[:, :N])
    t = jnp.maximum(acc * dis_col + b_ref[...], 0.0)
    val = jnp.sum(t * w2_ref[...], keepdims=True) * (1.0 / N) + b2_ref[...]
    out_ref[...] = val


@functools.cache
def _sc_kernels():
    mesh = plsc.VectorSubcoreMesh(
        core_axis_name="c", subcore_axis_name="s", num_cores=NC, num_subcores=NS
    )
    deg_kernel = pl.kernel(
        _deg_body,
        out_type=jax.ShapeDtypeStruct((NC * NPAD,), jnp.float32),
        mesh=mesh,
        scratch_types=[
            pltpu.VMEM((NCHUNK, C), jnp.int32),
            pltpu.VMEM((C,), jnp.float32),
            pltpu.VMEM_SHARED((NPAD,), jnp.float32),
        ],
    )
    agg_kernel = pl.kernel(
        _agg_body,
        out_type=jax.ShapeDtypeStruct((NC * NPAD, H), jnp.float32),
        mesh=mesh,
        compiler_params=pltpu.CompilerParams(use_tc_tiling_on_sc=False),
        scratch_types=[
            pltpu.VMEM((NCHUNK, C), jnp.int32),
            pltpu.VMEM((NCHUNK, C), jnp.int32),
            pltpu.VMEM((NB, C, H), jnp.float32),
            pltpu.VMEM_SHARED((NPAD, H), jnp.float32),
            pltpu.SemaphoreType.DMA((NB,)),
            pltpu.SemaphoreType.DMA((NB,)),
        ],
    )
    return deg_kernel, agg_kernel


_AR = np.arange(E_PAD - E, dtype=np.int32)
_PAD_SRC = jnp.asarray(_AR % N)
_PAD_DST = jnp.asarray(N + (_AR % (NPAD - N)))


def kernel(x, edge_index, W, b, W2, b2):
    src = edge_index[0].astype(jnp.int32)
    dst = edge_index[1].astype(jnp.int32)
    src_p = jnp.concatenate([src, _PAD_SRC]).reshape(NW, NCHUNK, C)
    dst_p = jnp.concatenate([dst, _PAD_DST]).reshape(NW, NCHUNK, C)

    zeros1 = jnp.zeros((NPAD,), jnp.float32)
    zerosh = jnp.zeros((NPAD, H), jnp.float32)

    deg_kernel, agg_kernel = _sc_kernels()
    degp = deg_kernel(dst_p, zeros1)
    degp2 = degp.reshape(NC, NPAD)

    g, dis = pl.pallas_call(
        _mmscale_body,
        out_shape=[
            jax.ShapeDtypeStruct((N, H), jnp.float32),
            jax.ShapeDtypeStruct((1, NPAD), jnp.float32),
        ],
    )(x, W, degp2)

    accp = agg_kernel(src_p, dst_p, g, zerosh)
    accp3 = accp.reshape(NC, NPAD, H)

    pooled = pl.pallas_call(
        _final_body,
        out_shape=jax.ShapeDtypeStruct((1, 1), jnp.float32),
    )(accp3, dis, b.reshape(1, H), W2.reshape(1, H), b2.reshape(1, 1))
    return pooled
